# baseline (speedup 1.0000x reference)
import jax
import jax.numpy as jnp
from jax import lax
from jax.experimental import pallas as pl
from jax.experimental.pallas import tpu as pltpu
from jax.experimental.pallas import tpu_sc as plsc

EPS = 1.1920929e-07
LOG_BASE = 9.210340371976184
S = 2048
D = 768
H = 12
DK = 64
E = 8
F = 2048
TQ = 256
T1 = 256
TS = 128
NPAIR = 2 * S
NSLOT = NPAIR + E * TS
NT = NSLOT // TS
NWORK = 32
PPW = NPAIR // NWORK
NEG = -1e30

bf16 = jnp.bfloat16
f32 = jnp.float32


def _rope(x, base):
    n = x.shape[0]
    pos = base + jax.lax.broadcasted_iota(jnp.int32, (n, 32), 0).astype(f32)
    fidx = jax.lax.broadcasted_iota(jnp.int32, (n, 32), 1).astype(f32)
    inv = jnp.exp(fidx * (-LOG_BASE / 32.0))
    ang = pos * inv
    c = jnp.cos(ang)
    sn = jnp.sin(ang)
    x1 = x[:, :32]
    x2 = x[:, 32:]
    return jnp.concatenate([x1 * c - x2 * sn, x2 * c + x1 * sn], axis=-1)


def _gelu(x):
    return 0.5 * x * (1.0 + jax.lax.erf(x * 0.7071067811865476))


def _qkv_kernel(x_ref, n1_ref, w_ref, o_ref):
    x = x_ref[...]
    h = x * jax.lax.rsqrt(jnp.mean(x * x, axis=-1, keepdims=True) + EPS) * n1_ref[...]
    o_ref[...] = jax.lax.dot_general(
        h.astype(bf16), w_ref[...], (((1,), (1,)), ((), ())),
        preferred_element_type=f32).astype(bf16)


def _run_qkv(xs, n1, wqkv_bf):
    return pl.pallas_call(
        _qkv_kernel,
        grid=(S // T1,),
        in_specs=[
            pl.BlockSpec((T1, D), lambda i: (i, 0)),
            pl.BlockSpec((1, D), lambda i: (0, 0)),
            pl.BlockSpec((3 * D, D), lambda i: (0, 0)),
        ],
        out_specs=pl.BlockSpec((T1, 3 * D), lambda i: (i, 0)),
        out_shape=jax.ShapeDtypeStruct((S, 3 * D), bf16),
        compiler_params=pltpu.CompilerParams(
            dimension_semantics=("arbitrary",)),
    )(xs, n1, wqkv_bf)


def _attn_one(q, kr, v, qt):
    s = jax.lax.dot_general(q, kr, (((1,), (1,)), ((), ())),
                            preferred_element_type=f32) * 0.125
    row = qt * TQ + jax.lax.broadcasted_iota(jnp.int32, (TQ, S), 0)
    col = jax.lax.broadcasted_iota(jnp.int32, (TQ, S), 1)
    s = jnp.where(col <= row, s, NEG)
    m = jnp.max(s, axis=-1, keepdims=True)
    p = jnp.exp(s - m)
    p = p / jnp.sum(p, axis=-1, keepdims=True)
    return jax.lax.dot_general(p.astype(bf16), v, (((1,), (0,)), ((), ())),
                               preferred_element_type=f32)


def _attn_kernel(q_ref, k_ref, v_ref, o_ref, kr_ref):
    qt = pl.program_id(1)

    @pl.when(qt == 0)
    def _():
        k = k_ref[...].astype(f32)
        kr_ref[...] = jnp.concatenate(
            [_rope(k[:, :DK], 0.0), _rope(k[:, DK:], 0.0)],
            axis=-1).astype(bf16)

    qf = q_ref[...].astype(f32)
    base = jnp.float32(qt) * TQ
    kr = kr_ref[...]
    v = v_ref[...]
    ol = _attn_one(_rope(qf[:, :DK], base).astype(bf16), kr[:, :DK], v[:, :DK], qt)
    orr = _attn_one(_rope(qf[:, DK:], base).astype(bf16), kr[:, DK:], v[:, DK:], qt)
    o_ref[...] = jnp.concatenate([ol, orr], axis=-1).astype(bf16)


def _run_attn(qkv):
    return pl.pallas_call(
        _attn_kernel,
        grid=(H // 2, S // TQ),
        in_specs=[
            pl.BlockSpec((TQ, 2 * DK), lambda h, qt: (qt, h)),
            pl.BlockSpec((S, 2 * DK), lambda h, qt: (0, H // 2 + h)),
            pl.BlockSpec((S, 2 * DK), lambda h, qt: (0, H + h)),
        ],
        out_specs=pl.BlockSpec((TQ, 2 * DK), lambda h, qt: (qt, h)),
        out_shape=jax.ShapeDtypeStruct((S, D), bf16),
        scratch_shapes=[pltpu.VMEM((S, 2 * DK), bf16)],
        compiler_params=pltpu.CompilerParams(
            dimension_semantics=("arbitrary", "arbitrary")),
    )(qkv, qkv, qkv)


def _post_kernel(x_ref, ao_ref, wo_ref, n2_ref, rw_ref,
                 x2_ref, hf_ref, tw_ref, ti_ref, aux_ref, ps_ref):
    tt = pl.program_id(0)
    x2 = x_ref[...] + jax.lax.dot_general(
        ao_ref[...], wo_ref[...], (((1,), (1,)), ((), ())),
        preferred_element_type=f32)
    x2_ref[...] = x2
    hf = x2 * jax.lax.rsqrt(jnp.mean(x2 * x2, axis=-1, keepdims=True) + EPS) * n2_ref[...]
    hf_ref[...] = hf
    logits = jax.lax.dot_general(hf, rw_ref[...], (((1,), (1,)), ((), ())),
                                 preferred_element_type=f32)
    colf = jax.lax.broadcasted_iota(jnp.int32, (T1, 128), 1)
    logits = jnp.where(colf < E, logits, NEG)
    m = jnp.max(logits, axis=-1, keepdims=True)
    p = jnp.exp(logits - m)
    probs = p / jnp.sum(p, axis=-1, keepdims=True)

    @pl.when(tt == 0)
    def _():
        ps_ref[...] = jnp.zeros_like(ps_ref)

    ps_ref[...] += jnp.sum(probs, axis=0, keepdims=True)

    @pl.when(tt == pl.num_programs(0) - 1)
    def _():
        mp = ps_ref[...] / jnp.float32(S)
        aux_ref[...] = jnp.sum(mp * mp, axis=-1, keepdims=True) * jnp.float32(E)

    m1 = jnp.max(probs, axis=-1, keepdims=True)
    i1 = jnp.min(jnp.where(probs == m1, colf, 128), axis=-1, keepdims=True)
    probs2 = jnp.where(colf == i1, -1.0, probs)
    m2 = jnp.max(probs2, axis=-1, keepdims=True)
    i2 = jnp.min(jnp.where(probs2 == m2, colf, 128), axis=-1, keepdims=True)
    tot = m1 + m2
    w1 = m1 / tot
    w2 = m2 / tot
    tw_ref[...] = (jnp.where(colf == 0, w1, 0.0)
                   + jnp.where(colf == 1, w2, 0.0))
    ti_ref[...] = (jnp.where(colf == 0, i1, 0)
                   + jnp.where(colf == 1, i2, 0))


def _run_post(xs, ao, wout_bf, n2, rw_pad):
    return pl.pallas_call(
        _post_kernel,
        grid=(S // T1,),
        in_specs=[
            pl.BlockSpec((T1, D), lambda i: (i, 0)),
            pl.BlockSpec((T1, D), lambda i: (i, 0)),
            pl.BlockSpec((D, D), lambda i: (0, 0)),
            pl.BlockSpec((1, D), lambda i: (0, 0)),
            pl.BlockSpec((128, D), lambda i: (0, 0)),
        ],
        out_specs=[
            pl.BlockSpec((T1, D), lambda i: (i, 0)),
            pl.BlockSpec((T1, D), lambda i: (i, 0)),
            pl.BlockSpec((T1, 128), lambda i: (i, 0)),
            pl.BlockSpec((T1, 128), lambda i: (i, 0)),
            pl.BlockSpec((1, 1), lambda i: (0, 0)),
        ],
        out_shape=[
            jax.ShapeDtypeStruct((S, D), f32),
            jax.ShapeDtypeStruct((S, D), f32),
            jax.ShapeDtypeStruct((S, 128), f32),
            jax.ShapeDtypeStruct((S, 128), jnp.int32),
            jax.ShapeDtypeStruct((1, 1), f32),
        ],
        scratch_shapes=[pltpu.VMEM((1, 128), f32)],
        compiler_params=pltpu.CompilerParams(
            dimension_semantics=("arbitrary",)),
    )(xs, ao, wout_bf, n2, rw_pad)


def _cumsum(x, axis):
    n = x.shape[axis]
    k = 1
    while k < n:
        if axis == 0:
            pad = jnp.zeros((k, x.shape[1]), x.dtype)
            x = x + jnp.concatenate([pad, x[:-k]], axis=0)
        else:
            pad = jnp.zeros((x.shape[0], k), x.dtype)
            x = x + jnp.concatenate([pad, x[:, :-k]], axis=1)
        k *= 2
    return x


def _meta_kernel(ti_ref, dest_ref, te_ref):
    col = jax.lax.broadcasted_iota(jnp.int32, (S, 128), 1)
    ti = ti_ref[...]
    i1 = ti[:, 0:1]
    i2 = ti[:, 1:2]
    oh1 = jnp.where((col == i1) & (col < E), 1, 0)
    oh2 = jnp.where((col == i2) & (col < E), 1, 0)
    c1 = _cumsum(oh1, 0)
    c2 = _cumsum(oh2, 0)
    cnt1 = c1[S - 1:S, :]
    cnt = cnt1 + c2[S - 1:S, :]
    padded = ((cnt + (TS - 1)) // TS) * TS
    offs = _cumsum(padded, 1) - padded
    d0 = jnp.sum(oh1 * (offs + c1 - 1), axis=1, keepdims=True)
    d1 = jnp.sum(oh2 * (offs + cnt1 + c2 - 1), axis=1, keepdims=True)
    dest_ref[...] = jnp.where(col == 0, d0, 0) + jnp.where(col == 1, d1, 0)
    rcol = jax.lax.broadcasted_iota(jnp.int32, (64, 128), 1)
    rrow = jax.lax.broadcasted_iota(jnp.int32, (64, 128), 0)
    ge = jnp.where((rcol < E) & (rrow * TS >= offs), 1, 0)
    te = jnp.sum(ge, axis=1, keepdims=True) - 1
    te_ref[...] = jnp.where(rcol == 0, te, 0)


def _run_meta(ti):
    return pl.pallas_call(
        _meta_kernel,
        grid=(1,),
        in_specs=[pl.BlockSpec((S, 128), lambda i: (0, 0))],
        out_specs=[
            pl.BlockSpec((S, 128), lambda i: (0, 0)),
            pl.BlockSpec((64, 128), lambda i: (0, 0)),
        ],
        out_shape=[
            jax.ShapeDtypeStruct((S, 128), jnp.int32),
            jax.ShapeDtypeStruct((64, 128), jnp.int32),
        ],
        compiler_params=pltpu.CompilerParams(
            dimension_semantics=("arbitrary",)),
    )(ti)


def _sc_mesh():
    return plsc.VectorSubcoreMesh(core_axis_name="c", subcore_axis_name="s")


def _dispatch_body(hf_hbm, dest_hbm, out_hbm, idx_v, rows_v, sem):
    wid = lax.axis_index("s") * 2 + lax.axis_index("c")
    base = wid * PPW
    tok = lax.rem(base, S)
    pltpu.sync_copy(dest_hbm.at[pl.ds(base, PPW)], idx_v)
    pltpu.sync_copy(hf_hbm.at[pl.ds(tok, PPW)], rows_v)
    pltpu.async_copy(rows_v, out_hbm.at[idx_v], sem).wait()


def _run_dispatch(hf, dest_flat):
    k = pl.kernel(
        _dispatch_body,
        mesh=_sc_mesh(),
        out_type=jax.ShapeDtypeStruct((NSLOT, D), f32),
        scratch_types=[
            pltpu.VMEM((PPW,), jnp.int32),
            pltpu.VMEM((PPW, D), f32),
            pltpu.SemaphoreType.DMA,
        ],
    )
    return k(hf, dest_flat)


def _moe_kernel(te_ref, rows_ref, w1_ref, w2_ref, o_ref):
    rows = rows_ref[...].astype(bf16)
    he = jax.lax.dot_general(rows, w1_ref[0], (((1,), (1,)), ((), ())),
                             preferred_element_type=f32)
    he = _gelu(he)
    o_ref[...] = jax.lax.dot_general(
        he.astype(bf16), w2_ref[0], (((1,), (1,)), ((), ())),
        preferred_element_type=f32)


def _run_moe(te, hf_sorted, w1_bf, w2_bf):
    grid_spec = pltpu.PrefetchScalarGridSpec(
        num_scalar_prefetch=1,
        grid=(NT,),
        in_specs=[
            pl.BlockSpec((TS, D), lambda i, te_s: (i, 0)),
            pl.BlockSpec((1, F, D), lambda i, te_s: (te_s[i], 0, 0)),
            pl.BlockSpec((1, D, F), lambda i, te_s: (te_s[i], 0, 0)),
        ],
        out_specs=pl.BlockSpec((TS, D), lambda i, te_s: (i, 0)),
    )
    return pl.pallas_call(
        _moe_kernel,
        grid_spec=grid_spec,
        out_shape=jax.ShapeDtypeStruct((NSLOT, D), f32),
        compiler_params=pltpu.CompilerParams(
            dimension_semantics=("arbitrary",)),
    )(te, hf_sorted, w1_bf, w2_bf)


def _gather_body(oe_hbm, dest_hbm, out_hbm, idx_v, rows_v, sem):
    wid = lax.axis_index("s") * 2 + lax.axis_index("c")
    base = wid * PPW
    pltpu.sync_copy(dest_hbm.at[pl.ds(base, PPW)], idx_v)
    pltpu.async_copy(oe_hbm.at[idx_v], rows_v, sem).wait()
    pltpu.sync_copy(rows_v, out_hbm.at[pl.ds(base, PPW)])


def _run_gather(oe_sorted, dest_flat):
    k = pl.kernel(
        _gather_body,
        mesh=_sc_mesh(),
        out_type=jax.ShapeDtypeStruct((NPAIR, D), f32),
        scratch_types=[
            pltpu.VMEM((PPW,), jnp.int32),
            pltpu.VMEM((PPW, D), f32),
            pltpu.SemaphoreType.DMA,
        ],
    )
    return k(oe_sorted, dest_flat)


def _combine_kernel(x2_ref, tw_ref, g0_ref, g1_ref, o_ref):
    tw = tw_ref[...]
    o_ref[...] = (x2_ref[...] + tw[:, 0:1] * g0_ref[...]
                  + tw[:, 1:2] * g1_ref[...])


def _run_combine(x2, tw, g01):
    return pl.pallas_call(
        _combine_kernel,
        grid=(S // T1,),
        in_specs=[
            pl.BlockSpec((T1, D), lambda i: (i, 0)),
            pl.BlockSpec((T1, 128), lambda i: (i, 0)),
            pl.BlockSpec((T1, D), lambda i: (i, 0)),
            pl.BlockSpec((T1, D), lambda i: (S // T1 + i, 0)),
        ],
        out_specs=pl.BlockSpec((T1, D), lambda i: (i, 0)),
        out_shape=jax.ShapeDtypeStruct((S, D), f32),
        compiler_params=pltpu.CompilerParams(
            dimension_semantics=("arbitrary",)),
    )(x2, tw, g01, g01)


def kernel(x, norm1_w, norm2_w, Wqkv, Wout, router_W, W1, W2):
    xs = x.reshape(S, D)
    n1 = norm1_w.reshape(1, D)
    n2 = norm2_w.reshape(1, D)
    wqkv_bf = Wqkv.astype(bf16)
    wout_bf = Wout.astype(bf16)
    w1_bf = W1.astype(bf16)
    w2_bf = W2.astype(bf16)
    rw_pad = jnp.zeros((128, D), f32).at[:E].set(router_W)

    qkv = _run_qkv(xs, n1, wqkv_bf)
    ao = _run_attn(qkv)
    x2, hf, tw, ti, aux = _run_post(xs, ao, wout_bf, n2, rw_pad)
    dest2, te2 = _run_meta(ti)
    dest_flat = jnp.concatenate([dest2[:, 0], dest2[:, 1]])
    te = te2[:NT, 0]
    hf_sorted = _run_dispatch(hf, dest_flat)
    oe_sorted = _run_moe(te, hf_sorted, w1_bf, w2_bf)
    g01 = _run_gather(oe_sorted, dest_flat)
    out = _run_combine(x2, tw, g01)
    return out.reshape(1, S, D), aux.reshape(())


# speedup vs baseline: 1.1061x; 1.1061x over previous
import jax
import jax.numpy as jnp
from jax import lax
from jax.experimental import pallas as pl
from jax.experimental.pallas import tpu as pltpu
from jax.experimental.pallas import tpu_sc as plsc

EPS = 1.1920929e-07
LOG_BASE = 9.210340371976184
S = 2048
D = 768
H = 12
DK = 64
E = 8
F = 2048
TQ = 256
T1 = 256
TS = 128
NPAIR = 2 * S
NSLOT = NPAIR + E * TS
NT = NSLOT // TS
NWORK = 32
PPW = NPAIR // NWORK
NEG = -1e30

bf16 = jnp.bfloat16
f32 = jnp.float32


def _rope(x, tab):
    c = tab[:, :32]
    sn = tab[:, 32:]
    x1 = x[:, :32]
    x2 = x[:, 32:]
    return jnp.concatenate([x1 * c - x2 * sn, x2 * c + x1 * sn], axis=-1)


def _gelu(x):
    return 0.5 * x * (1.0 + jax.lax.erf(x * 0.7071067811865476))


def _qkv_kernel(x_ref, n1_ref, w_ref, o_ref, tab_ref):
    x = x_ref[...]
    h = x * jax.lax.rsqrt(jnp.mean(x * x, axis=-1, keepdims=True) + EPS) * n1_ref[...]
    o_ref[...] = jax.lax.dot_general(
        h.astype(bf16), w_ref[...], (((1,), (1,)), ((), ())),
        preferred_element_type=f32).astype(bf16)
    pos = (pl.program_id(0) * T1
           + jax.lax.broadcasted_iota(jnp.int32, (T1, 32), 0)).astype(f32)
    fidx = jax.lax.broadcasted_iota(jnp.int32, (T1, 32), 1).astype(f32)
    ang = pos * jnp.exp(fidx * (-LOG_BASE / 32.0))
    tab_ref[...] = jnp.concatenate([jnp.cos(ang), jnp.sin(ang)], axis=-1)


def _run_qkv(xs, n1, wqkv_bf):
    return pl.pallas_call(
        _qkv_kernel,
        grid=(S // T1,),
        in_specs=[
            pl.BlockSpec((T1, D), lambda i: (i, 0)),
            pl.BlockSpec((1, D), lambda i: (0, 0)),
            pl.BlockSpec((3 * D, D), lambda i: (0, 0)),
        ],
        out_specs=[
            pl.BlockSpec((T1, 3 * D), lambda i: (i, 0)),
            pl.BlockSpec((T1, 2 * 32), lambda i: (i, 0)),
        ],
        out_shape=[
            jax.ShapeDtypeStruct((S, 3 * D), bf16),
            jax.ShapeDtypeStruct((S, 2 * 32), f32),
        ],
        compiler_params=pltpu.CompilerParams(
            dimension_semantics=("arbitrary",)),
    )(xs, n1, wqkv_bf)


def _attn_one(q, kr, v, qt):
    s = jax.lax.dot_general(q, kr, (((1,), (1,)), ((), ())),
                            preferred_element_type=f32) * 0.125
    row = qt * TQ + jax.lax.broadcasted_iota(jnp.int32, (TQ, S), 0)
    col = jax.lax.broadcasted_iota(jnp.int32, (TQ, S), 1)
    s = jnp.where(col <= row, s, NEG)
    m = jnp.max(s, axis=-1, keepdims=True)
    p = jnp.exp(s - m)
    p = p / jnp.sum(p, axis=-1, keepdims=True)
    return jax.lax.dot_general(p.astype(bf16), v, (((1,), (0,)), ((), ())),
                               preferred_element_type=f32)


def _attn_kernel(q_ref, k_ref, v_ref, tq_ref, tk_ref, o_ref, kr_ref):
    qt = pl.program_id(1)

    @pl.when(qt == 0)
    def _():
        k = k_ref[...].astype(f32)
        tk = tk_ref[...]
        kr_ref[...] = jnp.concatenate(
            [_rope(k[:, :DK], tk), _rope(k[:, DK:], tk)],
            axis=-1).astype(bf16)

    qf = q_ref[...].astype(f32)
    tq = tq_ref[...]
    kr = kr_ref[...]
    v = v_ref[...]
    ol = _attn_one(_rope(qf[:, :DK], tq).astype(bf16), kr[:, :DK], v[:, :DK], qt)
    orr = _attn_one(_rope(qf[:, DK:], tq).astype(bf16), kr[:, DK:], v[:, DK:], qt)
    o_ref[...] = jnp.concatenate([ol, orr], axis=-1).astype(bf16)


def _run_attn(qkv, tab):
    return pl.pallas_call(
        _attn_kernel,
        grid=(H // 2, S // TQ),
        in_specs=[
            pl.BlockSpec((TQ, 2 * DK), lambda h, qt: (qt, h)),
            pl.BlockSpec((S, 2 * DK), lambda h, qt: (0, H // 2 + h)),
            pl.BlockSpec((S, 2 * DK), lambda h, qt: (0, H + h)),
            pl.BlockSpec((TQ, 2 * 32), lambda h, qt: (qt, 0)),
            pl.BlockSpec((S, 2 * 32), lambda h, qt: (0, 0)),
        ],
        out_specs=pl.BlockSpec((TQ, 2 * DK), lambda h, qt: (qt, h)),
        out_shape=jax.ShapeDtypeStruct((S, D), bf16),
        scratch_shapes=[pltpu.VMEM((S, 2 * DK), bf16)],
        compiler_params=pltpu.CompilerParams(
            dimension_semantics=("arbitrary", "arbitrary")),
    )(qkv, qkv, qkv, tab, tab)


def _post_kernel(x_ref, ao_ref, wo_ref, n2_ref, rw_ref,
                 x2_ref, hf_ref, tw_ref, ti_ref, aux_ref, ps_ref):
    tt = pl.program_id(0)
    x2 = x_ref[...] + jax.lax.dot_general(
        ao_ref[...], wo_ref[...], (((1,), (1,)), ((), ())),
        preferred_element_type=f32)
    x2_ref[...] = x2
    hf = x2 * jax.lax.rsqrt(jnp.mean(x2 * x2, axis=-1, keepdims=True) + EPS) * n2_ref[...]
    hf_ref[...] = hf
    logits = jax.lax.dot_general(hf, rw_ref[...], (((1,), (1,)), ((), ())),
                                 preferred_element_type=f32)
    colf = jax.lax.broadcasted_iota(jnp.int32, (T1, 128), 1)
    logits = jnp.where(colf < E, logits, NEG)
    m = jnp.max(logits, axis=-1, keepdims=True)
    p = jnp.exp(logits - m)
    probs = p / jnp.sum(p, axis=-1, keepdims=True)

    @pl.when(tt == 0)
    def _():
        ps_ref[...] = jnp.zeros_like(ps_ref)

    ps_ref[...] += jnp.sum(probs, axis=0, keepdims=True)

    @pl.when(tt == pl.num_programs(0) - 1)
    def _():
        mp = ps_ref[...] / jnp.float32(S)
        aux_ref[...] = jnp.sum(mp * mp, axis=-1, keepdims=True) * jnp.float32(E)

    m1 = jnp.max(probs, axis=-1, keepdims=True)
    i1 = jnp.min(jnp.where(probs == m1, colf, 128), axis=-1, keepdims=True)
    probs2 = jnp.where(colf == i1, -1.0, probs)
    m2 = jnp.max(probs2, axis=-1, keepdims=True)
    i2 = jnp.min(jnp.where(probs2 == m2, colf, 128), axis=-1, keepdims=True)
    tot = m1 + m2
    w1 = m1 / tot
    w2 = m2 / tot
    tw_ref[...] = (jnp.where(colf == 0, w1, 0.0)
                   + jnp.where(colf == 1, w2, 0.0))
    ti_ref[...] = (jnp.where(colf == 0, i1, 0)
                   + jnp.where(colf == 1, i2, 0))


def _run_post(xs, ao, wout_bf, n2, rw_pad):
    return pl.pallas_call(
        _post_kernel,
        grid=(S // T1,),
        in_specs=[
            pl.BlockSpec((T1, D), lambda i: (i, 0)),
            pl.BlockSpec((T1, D), lambda i: (i, 0)),
            pl.BlockSpec((D, D), lambda i: (0, 0)),
            pl.BlockSpec((1, D), lambda i: (0, 0)),
            pl.BlockSpec((128, D), lambda i: (0, 0)),
        ],
        out_specs=[
            pl.BlockSpec((T1, D), lambda i: (i, 0)),
            pl.BlockSpec((T1, D), lambda i: (i, 0)),
            pl.BlockSpec((T1, 128), lambda i: (i, 0)),
            pl.BlockSpec((T1, 128), lambda i: (i, 0)),
            pl.BlockSpec((1, 1), lambda i: (0, 0)),
        ],
        out_shape=[
            jax.ShapeDtypeStruct((S, D), f32),
            jax.ShapeDtypeStruct((S, D), f32),
            jax.ShapeDtypeStruct((S, 128), f32),
            jax.ShapeDtypeStruct((S, 128), jnp.int32),
            jax.ShapeDtypeStruct((1, 1), f32),
        ],
        scratch_shapes=[pltpu.VMEM((1, 128), f32)],
        compiler_params=pltpu.CompilerParams(
            dimension_semantics=("arbitrary",)),
    )(xs, ao, wout_bf, n2, rw_pad)


def _cumsum(x, axis):
    n = x.shape[axis]
    k = 1
    while k < n:
        if axis == 0:
            pad = jnp.zeros((k, x.shape[1]), x.dtype)
            x = x + jnp.concatenate([pad, x[:-k]], axis=0)
        else:
            pad = jnp.zeros((x.shape[0], k), x.dtype)
            x = x + jnp.concatenate([pad, x[:, :-k]], axis=1)
        k *= 2
    return x


def _meta_kernel(ti_ref, dest_ref, te_ref):
    col = jax.lax.broadcasted_iota(jnp.int32, (S, 128), 1)
    ti = ti_ref[...]
    i1 = ti[:, 0:1]
    i2 = ti[:, 1:2]
    oh1 = jnp.where((col == i1) & (col < E), 1, 0)
    oh2 = jnp.where((col == i2) & (col < E), 1, 0)
    c1 = _cumsum(oh1, 0)
    c2 = _cumsum(oh2, 0)
    cnt1 = c1[S - 1:S, :]
    cnt = cnt1 + c2[S - 1:S, :]
    padded = ((cnt + (TS - 1)) // TS) * TS
    offs = _cumsum(padded, 1) - padded
    d0 = jnp.sum(oh1 * (offs + c1 - 1), axis=1, keepdims=True)
    d1 = jnp.sum(oh2 * (offs + cnt1 + c2 - 1), axis=1, keepdims=True)
    dest_ref[...] = jnp.where(col == 0, d0, 0) + jnp.where(col == 1, d1, 0)
    rcol = jax.lax.broadcasted_iota(jnp.int32, (64, 128), 1)
    rrow = jax.lax.broadcasted_iota(jnp.int32, (64, 128), 0)
    ge = jnp.where((rcol < E) & (rrow * TS >= offs), 1, 0)
    te = jnp.sum(ge, axis=1, keepdims=True) - 1
    te_ref[...] = jnp.where(rcol == 0, te, 0)


def _run_meta(ti):
    return pl.pallas_call(
        _meta_kernel,
        grid=(1,),
        in_specs=[pl.BlockSpec((S, 128), lambda i: (0, 0))],
        out_specs=[
            pl.BlockSpec((S, 128), lambda i: (0, 0)),
            pl.BlockSpec((64, 128), lambda i: (0, 0)),
        ],
        out_shape=[
            jax.ShapeDtypeStruct((S, 128), jnp.int32),
            jax.ShapeDtypeStruct((64, 128), jnp.int32),
        ],
        compiler_params=pltpu.CompilerParams(
            dimension_semantics=("arbitrary",)),
    )(ti)


def _sc_mesh():
    return plsc.VectorSubcoreMesh(core_axis_name="c", subcore_axis_name="s")


def _dispatch_body(hf_hbm, dest_hbm, out_hbm, idx_v, rows_v, sem):
    wid = lax.axis_index("s") * 2 + lax.axis_index("c")
    base = wid * PPW
    tok = lax.rem(base, S)
    pltpu.sync_copy(dest_hbm.at[pl.ds(base, PPW)], idx_v)
    pltpu.sync_copy(hf_hbm.at[pl.ds(tok, PPW)], rows_v)
    pltpu.async_copy(rows_v, out_hbm.at[idx_v], sem).wait()


def _run_dispatch(hf, dest_flat):
    k = pl.kernel(
        _dispatch_body,
        mesh=_sc_mesh(),
        out_type=jax.ShapeDtypeStruct((NSLOT, D), f32),
        scratch_types=[
            pltpu.VMEM((PPW,), jnp.int32),
            pltpu.VMEM((PPW, D), f32),
            pltpu.SemaphoreType.DMA,
        ],
    )
    return k(hf, dest_flat)


def _moe_kernel(te_ref, rows_ref, w1_ref, w2_ref, o_ref):
    rows = rows_ref[...].astype(bf16)
    he = jax.lax.dot_general(rows, w1_ref[0], (((1,), (1,)), ((), ())),
                             preferred_element_type=f32)
    he = _gelu(he)
    o_ref[...] = jax.lax.dot_general(
        he.astype(bf16), w2_ref[0], (((1,), (1,)), ((), ())),
        preferred_element_type=f32)


def _run_moe(te, hf_sorted, w1_bf, w2_bf):
    grid_spec = pltpu.PrefetchScalarGridSpec(
        num_scalar_prefetch=1,
        grid=(NT,),
        in_specs=[
            pl.BlockSpec((TS, D), lambda i, te_s: (i, 0)),
            pl.BlockSpec((1, F, D), lambda i, te_s: (te_s[i], 0, 0)),
            pl.BlockSpec((1, D, F), lambda i, te_s: (te_s[i], 0, 0)),
        ],
        out_specs=pl.BlockSpec((TS, D), lambda i, te_s: (i, 0)),
    )
    return pl.pallas_call(
        _moe_kernel,
        grid_spec=grid_spec,
        out_shape=jax.ShapeDtypeStruct((NSLOT, D), f32),
        compiler_params=pltpu.CompilerParams(
            dimension_semantics=("arbitrary",)),
    )(te, hf_sorted, w1_bf, w2_bf)


def _gather_body(oe_hbm, dest_hbm, out_hbm, idx_v, rows_v, sem):
    wid = lax.axis_index("s") * 2 + lax.axis_index("c")
    base = wid * PPW
    pltpu.sync_copy(dest_hbm.at[pl.ds(base, PPW)], idx_v)
    pltpu.async_copy(oe_hbm.at[idx_v], rows_v, sem).wait()
    pltpu.sync_copy(rows_v, out_hbm.at[pl.ds(base, PPW)])


def _run_gather(oe_sorted, dest_flat):
    k = pl.kernel(
        _gather_body,
        mesh=_sc_mesh(),
        out_type=jax.ShapeDtypeStruct((NPAIR, D), f32),
        scratch_types=[
            pltpu.VMEM((PPW,), jnp.int32),
            pltpu.VMEM((PPW, D), f32),
            pltpu.SemaphoreType.DMA,
        ],
    )
    return k(oe_sorted, dest_flat)


def _combine_kernel(x2_ref, tw_ref, g0_ref, g1_ref, o_ref):
    tw = tw_ref[...]
    o_ref[...] = (x2_ref[...] + tw[:, 0:1] * g0_ref[...]
                  + tw[:, 1:2] * g1_ref[...])


def _run_combine(x2, tw, g01):
    return pl.pallas_call(
        _combine_kernel,
        grid=(S // T1,),
        in_specs=[
            pl.BlockSpec((T1, D), lambda i: (i, 0)),
            pl.BlockSpec((T1, 128), lambda i: (i, 0)),
            pl.BlockSpec((T1, D), lambda i: (i, 0)),
            pl.BlockSpec((T1, D), lambda i: (S // T1 + i, 0)),
        ],
        out_specs=pl.BlockSpec((T1, D), lambda i: (i, 0)),
        out_shape=jax.ShapeDtypeStruct((S, D), f32),
        compiler_params=pltpu.CompilerParams(
            dimension_semantics=("arbitrary",)),
    )(x2, tw, g01, g01)


def kernel(x, norm1_w, norm2_w, Wqkv, Wout, router_W, W1, W2):
    xs = x.reshape(S, D)
    n1 = norm1_w.reshape(1, D)
    n2 = norm2_w.reshape(1, D)
    wqkv_bf = Wqkv.astype(bf16)
    wout_bf = Wout.astype(bf16)
    w1_bf = W1.astype(bf16)
    w2_bf = W2.astype(bf16)
    rw_pad = jnp.zeros((128, D), f32).at[:E].set(router_W)

    qkv, tab = _run_qkv(xs, n1, wqkv_bf)
    ao = _run_attn(qkv, tab)
    x2, hf, tw, ti, aux = _run_post(xs, ao, wout_bf, n2, rw_pad)
    dest2, te2 = _run_meta(ti)
    dest_flat = jnp.concatenate([dest2[:, 0], dest2[:, 1]])
    te = te2[:NT, 0]
    hf_sorted = _run_dispatch(hf, dest_flat)
    oe_sorted = _run_moe(te, hf_sorted, w1_bf, w2_bf)
    g01 = _run_gather(oe_sorted, dest_flat)
    out = _run_combine(x2, tw, g01)
    return out.reshape(1, S, D), aux.reshape(())


# speedup vs baseline: 1.2024x; 1.0870x over previous
import jax
import jax.numpy as jnp
from jax import lax
from jax.experimental import pallas as pl
from jax.experimental.pallas import tpu as pltpu
from jax.experimental.pallas import tpu_sc as plsc

EPS = 1.1920929e-07
LOG_BASE = 9.210340371976184
S = 2048
D = 768
H = 12
DK = 64
E = 8
F = 2048
TQ = 256
T1 = 256
TS = 128
NPAIR = 2 * S
NSLOT = NPAIR + E * TS
NT = NSLOT // TS
NWORK = 32
PPW = NPAIR // NWORK
NEG = -1e30

bf16 = jnp.bfloat16
f32 = jnp.float32


def _rope2(x, cc, ss):
    partner = jnp.concatenate(
        [x[:, 32:64], x[:, :32], x[:, 96:128], x[:, 64:96]], axis=-1)
    return x * cc + partner * ss


def _gelu(x):
    return 0.5 * x * (1.0 + jax.lax.erf(x * 0.7071067811865476))


def _qkv_kernel(x_ref, n1_ref, w_ref, o_ref, tab_ref):
    x = x_ref[...]
    h = x * jax.lax.rsqrt(jnp.mean(x * x, axis=-1, keepdims=True) + EPS) * n1_ref[...]
    o_ref[...] = jax.lax.dot_general(
        h.astype(bf16), w_ref[...], (((1,), (1,)), ((), ())),
        preferred_element_type=f32).astype(bf16)
    pos = (pl.program_id(0) * T1
           + jax.lax.broadcasted_iota(jnp.int32, (T1, 32), 0)).astype(f32)
    fidx = jax.lax.broadcasted_iota(jnp.int32, (T1, 32), 1).astype(f32)
    ang = pos * jnp.exp(fidx * (-LOG_BASE / 32.0))
    c = jnp.cos(ang)
    sn = jnp.sin(ang)
    tab_ref[...] = jnp.concatenate([c, c, c, c, -sn, sn, -sn, sn], axis=-1)


def _run_qkv(xs, n1, wqkv_bf):
    return pl.pallas_call(
        _qkv_kernel,
        grid=(S // T1,),
        in_specs=[
            pl.BlockSpec((T1, D), lambda i: (i, 0)),
            pl.BlockSpec((1, D), lambda i: (0, 0)),
            pl.BlockSpec((3 * D, D), lambda i: (0, 0)),
        ],
        out_specs=[
            pl.BlockSpec((T1, 3 * D), lambda i: (i, 0)),
            pl.BlockSpec((T1, 256), lambda i: (i, 0)),
        ],
        out_shape=[
            jax.ShapeDtypeStruct((S, 3 * D), bf16),
            jax.ShapeDtypeStruct((S, 256), f32),
        ],
        compiler_params=pltpu.CompilerParams(
            dimension_semantics=("arbitrary",)),
    )(xs, n1, wqkv_bf)


TK = 512


def _attn_kernel(q_ref, k_ref, v_ref, tq_ref, tk_ref, o_ref, kr_ref):
    qt = pl.program_id(1)

    @pl.when(qt == 0)
    def _():
        k = k_ref[...].astype(f32)
        tk = tk_ref[...]
        kr_ref[...] = _rope2(k, tk[:, :128], tk[:, 128:]).astype(bf16)

    tq = tq_ref[...]
    q = _rope2(q_ref[...].astype(f32), tq[:, :128], tq[:, 128:]).astype(bf16)
    q0 = q[:, :DK]
    q1 = q[:, DK:]
    row = qt * TQ + jax.lax.broadcasted_iota(jnp.int32, (TQ, TK), 0)
    colb = jax.lax.broadcasted_iota(jnp.int32, (TQ, TK), 1)

    def body(kt, carry):
        o0, o1, l0, l1 = carry
        kc = kr_ref[pl.ds(kt * TK, TK), :]
        vc = v_ref[pl.ds(kt * TK, TK), :]
        col = kt * TK + colb

        def one(qh, hs):
            sc = jax.lax.dot_general(
                qh, kc[:, hs:hs + DK], (((1,), (1,)), ((), ())),
                preferred_element_type=f32)
            p = jnp.exp(jnp.where(col <= row, sc * 0.125 - 20.0, NEG))
            o = jax.lax.dot_general(
                p.astype(bf16), vc[:, hs:hs + DK], (((1,), (0,)), ((), ())),
                preferred_element_type=f32)
            return o, jnp.sum(p, axis=-1, keepdims=True)

        a0, b0 = one(q0, 0)
        a1, b1 = one(q1, DK)
        return o0 + a0, o1 + a1, l0 + b0, l1 + b1

    z = jnp.zeros((TQ, DK), f32)
    zl = jnp.zeros((TQ, 1), f32)
    o0, o1, l0, l1 = jax.lax.fori_loop(
        0, qt // (TK // TQ) + 1, body, (z, z, zl, zl))
    o_ref[...] = jnp.concatenate([o0 / l0, o1 / l1], axis=-1).astype(bf16)


def _run_attn(qkv, tab):
    return pl.pallas_call(
        _attn_kernel,
        grid=(H // 2, S // TQ),
        in_specs=[
            pl.BlockSpec((TQ, 2 * DK), lambda h, qt: (qt, h)),
            pl.BlockSpec((S, 2 * DK), lambda h, qt: (0, H // 2 + h)),
            pl.BlockSpec((S, 2 * DK), lambda h, qt: (0, H + h)),
            pl.BlockSpec((TQ, 256), lambda h, qt: (qt, 0)),
            pl.BlockSpec((S, 256), lambda h, qt: (0, 0)),
        ],
        out_specs=pl.BlockSpec((TQ, 2 * DK), lambda h, qt: (qt, h)),
        out_shape=jax.ShapeDtypeStruct((S, D), bf16),
        scratch_shapes=[pltpu.VMEM((S, 2 * DK), bf16)],
        compiler_params=pltpu.CompilerParams(
            dimension_semantics=("arbitrary", "arbitrary")),
    )(qkv, qkv, qkv, tab, tab)


def _post_kernel(x_ref, ao_ref, wo_ref, n2_ref, rw_ref,
                 x2_ref, hf_ref, tw_ref, ti_ref, aux_ref, ps_ref):
    tt = pl.program_id(0)
    x2 = x_ref[...] + jax.lax.dot_general(
        ao_ref[...], wo_ref[...], (((1,), (1,)), ((), ())),
        preferred_element_type=f32)
    x2_ref[...] = x2
    hf = x2 * jax.lax.rsqrt(jnp.mean(x2 * x2, axis=-1, keepdims=True) + EPS) * n2_ref[...]
    hf_ref[...] = hf
    logits = jax.lax.dot_general(hf, rw_ref[...], (((1,), (1,)), ((), ())),
                                 preferred_element_type=f32)
    colf = jax.lax.broadcasted_iota(jnp.int32, (T1, 128), 1)
    logits = jnp.where(colf < E, logits, NEG)
    m = jnp.max(logits, axis=-1, keepdims=True)
    p = jnp.exp(logits - m)
    probs = p / jnp.sum(p, axis=-1, keepdims=True)

    @pl.when(tt == 0)
    def _():
        ps_ref[...] = jnp.zeros_like(ps_ref)

    ps_ref[...] += jnp.sum(probs, axis=0, keepdims=True)

    @pl.when(tt == pl.num_programs(0) - 1)
    def _():
        mp = ps_ref[...] / jnp.float32(S)
        aux_ref[...] = jnp.sum(mp * mp, axis=-1, keepdims=True) * jnp.float32(E)

    m1 = jnp.max(probs, axis=-1, keepdims=True)
    i1 = jnp.min(jnp.where(probs == m1, colf, 128), axis=-1, keepdims=True)
    probs2 = jnp.where(colf == i1, -1.0, probs)
    m2 = jnp.max(probs2, axis=-1, keepdims=True)
    i2 = jnp.min(jnp.where(probs2 == m2, colf, 128), axis=-1, keepdims=True)
    tot = m1 + m2
    w1 = m1 / tot
    w2 = m2 / tot
    tw_ref[...] = (jnp.where(colf == 0, w1, 0.0)
                   + jnp.where(colf == 1, w2, 0.0))
    ti_ref[...] = (jnp.where(colf == 0, i1, 0)
                   + jnp.where(colf == 1, i2, 0))


def _run_post(xs, ao, wout_bf, n2, rw_pad):
    return pl.pallas_call(
        _post_kernel,
        grid=(S // T1,),
        in_specs=[
            pl.BlockSpec((T1, D), lambda i: (i, 0)),
            pl.BlockSpec((T1, D), lambda i: (i, 0)),
            pl.BlockSpec((D, D), lambda i: (0, 0)),
            pl.BlockSpec((1, D), lambda i: (0, 0)),
            pl.BlockSpec((128, D), lambda i: (0, 0)),
        ],
        out_specs=[
            pl.BlockSpec((T1, D), lambda i: (i, 0)),
            pl.BlockSpec((T1, D), lambda i: (i, 0)),
            pl.BlockSpec((T1, 128), lambda i: (i, 0)),
            pl.BlockSpec((T1, 128), lambda i: (i, 0)),
            pl.BlockSpec((1, 1), lambda i: (0, 0)),
        ],
        out_shape=[
            jax.ShapeDtypeStruct((S, D), f32),
            jax.ShapeDtypeStruct((S, D), f32),
            jax.ShapeDtypeStruct((S, 128), f32),
            jax.ShapeDtypeStruct((S, 128), jnp.int32),
            jax.ShapeDtypeStruct((1, 1), f32),
        ],
        scratch_shapes=[pltpu.VMEM((1, 128), f32)],
        compiler_params=pltpu.CompilerParams(
            dimension_semantics=("arbitrary",)),
    )(xs, ao, wout_bf, n2, rw_pad)


def _cumsum(x, axis):
    n = x.shape[axis]
    k = 1
    while k < n:
        if axis == 0:
            pad = jnp.zeros((k, x.shape[1]), x.dtype)
            x = x + jnp.concatenate([pad, x[:-k]], axis=0)
        else:
            pad = jnp.zeros((x.shape[0], k), x.dtype)
            x = x + jnp.concatenate([pad, x[:, :-k]], axis=1)
        k *= 2
    return x


def _meta_kernel(ti_ref, dest_ref, te_ref):
    col = jax.lax.broadcasted_iota(jnp.int32, (S, 128), 1)
    ti = ti_ref[...]
    i1 = ti[:, 0:1]
    i2 = ti[:, 1:2]
    oh1 = jnp.where((col == i1) & (col < E), 1, 0)
    oh2 = jnp.where((col == i2) & (col < E), 1, 0)
    c1 = _cumsum(oh1, 0)
    c2 = _cumsum(oh2, 0)
    cnt1 = c1[S - 1:S, :]
    cnt = cnt1 + c2[S - 1:S, :]
    padded = ((cnt + (TS - 1)) // TS) * TS
    offs = _cumsum(padded, 1) - padded
    d0 = jnp.sum(oh1 * (offs + c1 - 1), axis=1, keepdims=True)
    d1 = jnp.sum(oh2 * (offs + cnt1 + c2 - 1), axis=1, keepdims=True)
    dest_ref[...] = jnp.where(col == 0, d0, 0) + jnp.where(col == 1, d1, 0)
    rcol = jax.lax.broadcasted_iota(jnp.int32, (64, 128), 1)
    rrow = jax.lax.broadcasted_iota(jnp.int32, (64, 128), 0)
    ge = jnp.where((rcol < E) & (rrow * TS >= offs), 1, 0)
    te = jnp.sum(ge, axis=1, keepdims=True) - 1
    te_ref[...] = jnp.where(rcol == 0, te, 0)


def _run_meta(ti):
    return pl.pallas_call(
        _meta_kernel,
        grid=(1,),
        in_specs=[pl.BlockSpec((S, 128), lambda i: (0, 0))],
        out_specs=[
            pl.BlockSpec((S, 128), lambda i: (0, 0)),
            pl.BlockSpec((64, 128), lambda i: (0, 0)),
        ],
        out_shape=[
            jax.ShapeDtypeStruct((S, 128), jnp.int32),
            jax.ShapeDtypeStruct((64, 128), jnp.int32),
        ],
        compiler_params=pltpu.CompilerParams(
            dimension_semantics=("arbitrary",)),
    )(ti)


def _sc_mesh():
    return plsc.VectorSubcoreMesh(core_axis_name="c", subcore_axis_name="s")


def _dispatch_body(hf_hbm, dest_hbm, out_hbm, idx_v, rows_v, sem):
    wid = lax.axis_index("s") * 2 + lax.axis_index("c")
    base = wid * PPW
    tok = lax.rem(base, S)
    pltpu.sync_copy(dest_hbm.at[pl.ds(base, PPW)], idx_v)
    pltpu.sync_copy(hf_hbm.at[pl.ds(tok, PPW)], rows_v)
    pltpu.async_copy(rows_v, out_hbm.at[idx_v], sem).wait()


def _run_dispatch(hf, dest_flat):
    k = pl.kernel(
        _dispatch_body,
        mesh=_sc_mesh(),
        out_type=jax.ShapeDtypeStruct((NSLOT, D), f32),
        scratch_types=[
            pltpu.VMEM((PPW,), jnp.int32),
            pltpu.VMEM((PPW, D), f32),
            pltpu.SemaphoreType.DMA,
        ],
    )
    return k(hf, dest_flat)


def _moe_kernel(te_ref, rows_ref, w1_ref, w2_ref, o_ref):
    rows = rows_ref[...].astype(bf16)
    he = jax.lax.dot_general(rows, w1_ref[0], (((1,), (1,)), ((), ())),
                             preferred_element_type=f32)
    he = _gelu(he)
    o_ref[...] = jax.lax.dot_general(
        he.astype(bf16), w2_ref[0], (((1,), (1,)), ((), ())),
        preferred_element_type=f32)


def _run_moe(te, hf_sorted, w1_bf, w2_bf):
    grid_spec = pltpu.PrefetchScalarGridSpec(
        num_scalar_prefetch=1,
        grid=(NT,),
        in_specs=[
            pl.BlockSpec((TS, D), lambda i, te_s: (i, 0)),
            pl.BlockSpec((1, F, D), lambda i, te_s: (te_s[i], 0, 0)),
            pl.BlockSpec((1, D, F), lambda i, te_s: (te_s[i], 0, 0)),
        ],
        out_specs=pl.BlockSpec((TS, D), lambda i, te_s: (i, 0)),
    )
    return pl.pallas_call(
        _moe_kernel,
        grid_spec=grid_spec,
        out_shape=jax.ShapeDtypeStruct((NSLOT, D), f32),
        compiler_params=pltpu.CompilerParams(
            dimension_semantics=("arbitrary",)),
    )(te, hf_sorted, w1_bf, w2_bf)


def _gather_body(oe_hbm, dest_hbm, out_hbm, idx_v, rows_v, sem):
    wid = lax.axis_index("s") * 2 + lax.axis_index("c")
    base = wid * PPW
    pltpu.sync_copy(dest_hbm.at[pl.ds(base, PPW)], idx_v)
    pltpu.async_copy(oe_hbm.at[idx_v], rows_v, sem).wait()
    pltpu.sync_copy(rows_v, out_hbm.at[pl.ds(base, PPW)])


def _run_gather(oe_sorted, dest_flat):
    k = pl.kernel(
        _gather_body,
        mesh=_sc_mesh(),
        out_type=jax.ShapeDtypeStruct((NPAIR, D), f32),
        scratch_types=[
            pltpu.VMEM((PPW,), jnp.int32),
            pltpu.VMEM((PPW, D), f32),
            pltpu.SemaphoreType.DMA,
        ],
    )
    return k(oe_sorted, dest_flat)


def _combine_kernel(x2_ref, tw_ref, g0_ref, g1_ref, o_ref):
    tw = tw_ref[...]
    o_ref[...] = (x2_ref[...] + tw[:, 0:1] * g0_ref[...]
                  + tw[:, 1:2] * g1_ref[...])


def _run_combine(x2, tw, g01):
    return pl.pallas_call(
        _combine_kernel,
        grid=(S // T1,),
        in_specs=[
            pl.BlockSpec((T1, D), lambda i: (i, 0)),
            pl.BlockSpec((T1, 128), lambda i: (i, 0)),
            pl.BlockSpec((T1, D), lambda i: (i, 0)),
            pl.BlockSpec((T1, D), lambda i: (S // T1 + i, 0)),
        ],
        out_specs=pl.BlockSpec((T1, D), lambda i: (i, 0)),
        out_shape=jax.ShapeDtypeStruct((S, D), f32),
        compiler_params=pltpu.CompilerParams(
            dimension_semantics=("arbitrary",)),
    )(x2, tw, g01, g01)


def kernel(x, norm1_w, norm2_w, Wqkv, Wout, router_W, W1, W2):
    xs = x.reshape(S, D)
    n1 = norm1_w.reshape(1, D)
    n2 = norm2_w.reshape(1, D)
    wqkv_bf = Wqkv.astype(bf16)
    wout_bf = Wout.astype(bf16)
    w1_bf = W1.astype(bf16)
    w2_bf = W2.astype(bf16)
    rw_pad = jnp.zeros((128, D), f32).at[:E].set(router_W)

    qkv, tab = _run_qkv(xs, n1, wqkv_bf)
    ao = _run_attn(qkv, tab)
    x2, hf, tw, ti, aux = _run_post(xs, ao, wout_bf, n2, rw_pad)
    dest2, te2 = _run_meta(ti)
    dest_flat = jnp.concatenate([dest2[:, 0], dest2[:, 1]])
    te = te2[:NT, 0]
    hf_sorted = _run_dispatch(hf, dest_flat)
    oe_sorted = _run_moe(te, hf_sorted, w1_bf, w2_bf)
    g01 = _run_gather(oe_sorted, dest_flat)
    out = _run_combine(x2, tw, g01)
    return out.reshape(1, S, D), aux.reshape(())


# speedup vs baseline: 1.3985x; 1.1631x over previous
import jax
import jax.numpy as jnp
from jax import lax
from jax.experimental import pallas as pl
from jax.experimental.pallas import tpu as pltpu
from jax.experimental.pallas import tpu_sc as plsc

EPS = 1.1920929e-07
LOG_BASE = 9.210340371976184
S = 2048
D = 768
H = 12
DK = 64
E = 8
F = 2048
TQ = 512
T1 = 256
TS = 128
NPAIR = 2 * S
NSLOT = NPAIR + E * TS
NT = NSLOT // TS
NWORK = 32
PPW = NPAIR // NWORK
NEG = -1e30

bf16 = jnp.bfloat16
f32 = jnp.float32


def _rope2(x, cc, ss):
    partner = jnp.concatenate(
        [x[:, 32:64], x[:, :32], x[:, 96:128], x[:, 64:96]], axis=-1)
    return x * cc + partner * ss


def _gelu(x):
    return 0.5 * x * (1.0 + jax.lax.erf(x * 0.7071067811865476))


def _qkv_kernel(x_ref, n1_ref, w_ref, o_ref, tab_ref):
    x = x_ref[...]
    h = x * jax.lax.rsqrt(jnp.mean(x * x, axis=-1, keepdims=True) + EPS) * n1_ref[...]
    o_ref[...] = jax.lax.dot_general(
        h.astype(bf16), w_ref[...], (((1,), (1,)), ((), ())),
        preferred_element_type=f32).astype(bf16)
    pos = (pl.program_id(0) * T1
           + jax.lax.broadcasted_iota(jnp.int32, (T1, 32), 0)).astype(f32)
    fidx = jax.lax.broadcasted_iota(jnp.int32, (T1, 32), 1).astype(f32)
    ang = pos * jnp.exp(fidx * (-LOG_BASE / 32.0))
    c = jnp.cos(ang)
    sn = jnp.sin(ang)
    tab_ref[...] = jnp.concatenate([c, c, c, c, -sn, sn, -sn, sn], axis=-1)


def _run_qkv(xs, n1, wqkv_bf):
    return pl.pallas_call(
        _qkv_kernel,
        grid=(S // T1,),
        in_specs=[
            pl.BlockSpec((T1, D), lambda i: (i, 0)),
            pl.BlockSpec((1, D), lambda i: (0, 0)),
            pl.BlockSpec((3 * D, D), lambda i: (0, 0)),
        ],
        out_specs=[
            pl.BlockSpec((T1, 3 * D), lambda i: (i, 0)),
            pl.BlockSpec((T1, 256), lambda i: (i, 0)),
        ],
        out_shape=[
            jax.ShapeDtypeStruct((S, 3 * D), bf16),
            jax.ShapeDtypeStruct((S, 256), f32),
        ],
        compiler_params=pltpu.CompilerParams(
            dimension_semantics=("arbitrary",)),
    )(xs, n1, wqkv_bf)


TK = 512


def _attn_kernel(q_ref, k_ref, v_ref, tq_ref, tk_ref, o_ref, kr_ref):
    qt = pl.program_id(1)

    @pl.when(qt == 0)
    def _():
        k = k_ref[...].astype(f32)
        tk = tk_ref[...]
        kr_ref[...] = _rope2(k, tk[:, :128], tk[:, 128:]).astype(bf16)

    tq = tq_ref[...]
    q = _rope2(q_ref[...].astype(f32), tq[:, :128], tq[:, 128:]).astype(bf16)
    q0 = q[:, :DK]
    q1 = q[:, DK:]
    row = qt * TQ + jax.lax.broadcasted_iota(jnp.int32, (TQ, TK), 0)
    colb = jax.lax.broadcasted_iota(jnp.int32, (TQ, TK), 1)

    def body(kt, carry):
        o0, o1, l0, l1 = carry
        kc = kr_ref[pl.ds(kt * TK, TK), :]
        vc = v_ref[pl.ds(kt * TK, TK), :]
        col = kt * TK + colb

        def one(qh, hs):
            sc = jax.lax.dot_general(
                qh, kc[:, hs:hs + DK], (((1,), (1,)), ((), ())),
                preferred_element_type=f32)
            p = jnp.exp(jnp.where(col <= row, sc * 0.125 - 20.0, NEG))
            o = jax.lax.dot_general(
                p.astype(bf16), vc[:, hs:hs + DK], (((1,), (0,)), ((), ())),
                preferred_element_type=f32)
            return o, jnp.sum(p, axis=-1, keepdims=True)

        a0, b0 = one(q0, 0)
        a1, b1 = one(q1, DK)
        return o0 + a0, o1 + a1, l0 + b0, l1 + b1

    z = jnp.zeros((TQ, DK), f32)
    zl = jnp.zeros((TQ, 1), f32)
    o0, o1, l0, l1 = jax.lax.fori_loop(
        0, qt // (TK // TQ) + 1, body, (z, z, zl, zl))
    o_ref[...] = jnp.concatenate([o0 / l0, o1 / l1], axis=-1).astype(bf16)


def _run_attn(qkv, tab):
    return pl.pallas_call(
        _attn_kernel,
        grid=(H // 2, S // TQ),
        in_specs=[
            pl.BlockSpec((TQ, 2 * DK), lambda h, qt: (qt, h)),
            pl.BlockSpec((S, 2 * DK), lambda h, qt: (0, H // 2 + h)),
            pl.BlockSpec((S, 2 * DK), lambda h, qt: (0, H + h)),
            pl.BlockSpec((TQ, 256), lambda h, qt: (qt, 0)),
            pl.BlockSpec((S, 256), lambda h, qt: (0, 0)),
        ],
        out_specs=pl.BlockSpec((TQ, 2 * DK), lambda h, qt: (qt, h)),
        out_shape=jax.ShapeDtypeStruct((S, D), bf16),
        scratch_shapes=[pltpu.VMEM((S, 2 * DK), bf16)],
        compiler_params=pltpu.CompilerParams(
            dimension_semantics=("arbitrary", "arbitrary")),
    )(qkv, qkv, qkv, tab, tab)


def _post_kernel(x_ref, ao_ref, wo_ref, n2_ref, rw_ref,
                 x2_ref, hf_ref, tw_ref, ti_ref, aux_ref, ps_ref):
    tt = pl.program_id(0)
    x2 = x_ref[...] + jax.lax.dot_general(
        ao_ref[...], wo_ref[...], (((1,), (1,)), ((), ())),
        preferred_element_type=f32)
    x2_ref[...] = x2
    hf = x2 * jax.lax.rsqrt(jnp.mean(x2 * x2, axis=-1, keepdims=True) + EPS) * n2_ref[...]
    hf_ref[...] = hf
    logits = jax.lax.dot_general(hf, rw_ref[...], (((1,), (1,)), ((), ())),
                                 preferred_element_type=f32)
    colf = jax.lax.broadcasted_iota(jnp.int32, (T1, 128), 1)
    logits = jnp.where(colf < E, logits, NEG)
    m = jnp.max(logits, axis=-1, keepdims=True)
    p = jnp.exp(logits - m)
    probs = p / jnp.sum(p, axis=-1, keepdims=True)

    @pl.when(tt == 0)
    def _():
        ps_ref[...] = jnp.zeros_like(ps_ref)

    ps_ref[...] += jnp.sum(probs, axis=0, keepdims=True)

    @pl.when(tt == pl.num_programs(0) - 1)
    def _():
        mp = ps_ref[...] / jnp.float32(S)
        aux_ref[...] = jnp.sum(mp * mp, axis=-1, keepdims=True) * jnp.float32(E)

    m1 = jnp.max(probs, axis=-1, keepdims=True)
    i1 = jnp.min(jnp.where(probs == m1, colf, 128), axis=-1, keepdims=True)
    probs2 = jnp.where(colf == i1, -1.0, probs)
    m2 = jnp.max(probs2, axis=-1, keepdims=True)
    i2 = jnp.min(jnp.where(probs2 == m2, colf, 128), axis=-1, keepdims=True)
    tot = m1 + m2
    w1 = m1 / tot
    w2 = m2 / tot
    tw_ref[...] = (jnp.where(colf == 0, w1, 0.0)
                   + jnp.where(colf == 1, w2, 0.0))
    ti_ref[...] = (jnp.where(colf == 0, i1, 0)
                   + jnp.where(colf == 1, i2, 0))


def _run_post(xs, ao, wout_bf, n2, rw_pad):
    return pl.pallas_call(
        _post_kernel,
        grid=(S // T1,),
        in_specs=[
            pl.BlockSpec((T1, D), lambda i: (i, 0)),
            pl.BlockSpec((T1, D), lambda i: (i, 0)),
            pl.BlockSpec((D, D), lambda i: (0, 0)),
            pl.BlockSpec((1, D), lambda i: (0, 0)),
            pl.BlockSpec((128, D), lambda i: (0, 0)),
        ],
        out_specs=[
            pl.BlockSpec((T1, D), lambda i: (i, 0)),
            pl.BlockSpec((T1, D), lambda i: (i, 0)),
            pl.BlockSpec((T1, 128), lambda i: (i, 0)),
            pl.BlockSpec((T1, 128), lambda i: (i, 0)),
            pl.BlockSpec((1, 1), lambda i: (0, 0)),
        ],
        out_shape=[
            jax.ShapeDtypeStruct((S, D), f32),
            jax.ShapeDtypeStruct((S, D), f32),
            jax.ShapeDtypeStruct((S, 128), f32),
            jax.ShapeDtypeStruct((S, 128), jnp.int32),
            jax.ShapeDtypeStruct((1, 1), f32),
        ],
        scratch_shapes=[pltpu.VMEM((1, 128), f32)],
        compiler_params=pltpu.CompilerParams(
            dimension_semantics=("arbitrary",)),
    )(xs, ao, wout_bf, n2, rw_pad)


def _cumsum(x, axis):
    n = x.shape[axis]
    k = 1
    while k < n:
        if axis == 0:
            pad = jnp.zeros((k, x.shape[1]), x.dtype)
            x = x + jnp.concatenate([pad, x[:-k]], axis=0)
        else:
            pad = jnp.zeros((x.shape[0], k), x.dtype)
            x = x + jnp.concatenate([pad, x[:, :-k]], axis=1)
        k *= 2
    return x


def _meta_kernel(ti_ref, dest_ref, te_ref):
    col = jax.lax.broadcasted_iota(jnp.int32, (S, 128), 1)
    ti = ti_ref[...]
    i1 = ti[:, 0:1]
    i2 = ti[:, 1:2]
    oh1 = jnp.where((col == i1) & (col < E), 1, 0)
    oh2 = jnp.where((col == i2) & (col < E), 1, 0)
    c1 = _cumsum(oh1, 0)
    c2 = _cumsum(oh2, 0)
    cnt1 = c1[S - 1:S, :]
    cnt = cnt1 + c2[S - 1:S, :]
    padded = ((cnt + (TS - 1)) // TS) * TS
    offs = _cumsum(padded, 1) - padded
    d0 = jnp.sum(oh1 * (offs + c1 - 1), axis=1, keepdims=True)
    d1 = jnp.sum(oh2 * (offs + cnt1 + c2 - 1), axis=1, keepdims=True)
    dest_ref[...] = jnp.where(col == 0, d0, 0) + jnp.where(col == 1, d1, 0)
    rcol = jax.lax.broadcasted_iota(jnp.int32, (64, 128), 1)
    rrow = jax.lax.broadcasted_iota(jnp.int32, (64, 128), 0)
    ge = jnp.where((rcol < E) & (rrow * TS >= offs), 1, 0)
    te = jnp.sum(ge, axis=1, keepdims=True) - 1
    te_ref[...] = jnp.where(rcol == 0, te, 0)


def _run_meta(ti):
    return pl.pallas_call(
        _meta_kernel,
        grid=(1,),
        in_specs=[pl.BlockSpec((S, 128), lambda i: (0, 0))],
        out_specs=[
            pl.BlockSpec((S, 128), lambda i: (0, 0)),
            pl.BlockSpec((64, 128), lambda i: (0, 0)),
        ],
        out_shape=[
            jax.ShapeDtypeStruct((S, 128), jnp.int32),
            jax.ShapeDtypeStruct((64, 128), jnp.int32),
        ],
        compiler_params=pltpu.CompilerParams(
            dimension_semantics=("arbitrary",)),
    )(ti)


def _sc_mesh():
    return plsc.VectorSubcoreMesh(core_axis_name="c", subcore_axis_name="s")


def _dispatch_body(hf_hbm, dest_hbm, out_hbm, idx_v, rows_v, sem):
    wid = lax.axis_index("s") * 2 + lax.axis_index("c")
    base = wid * PPW
    tok = lax.rem(base, S)
    pltpu.sync_copy(dest_hbm.at[pl.ds(base, PPW)], idx_v)
    pltpu.sync_copy(hf_hbm.at[pl.ds(tok, PPW)], rows_v)
    pltpu.async_copy(rows_v, out_hbm.at[idx_v], sem).wait()


def _run_dispatch(hf, dest_flat):
    k = pl.kernel(
        _dispatch_body,
        mesh=_sc_mesh(),
        out_type=jax.ShapeDtypeStruct((NSLOT, D), f32),
        scratch_types=[
            pltpu.VMEM((PPW,), jnp.int32),
            pltpu.VMEM((PPW, D), f32),
            pltpu.SemaphoreType.DMA,
        ],
    )
    return k(hf, dest_flat)


def _moe_kernel(te_ref, rows_ref, w1_ref, w2_ref, o_ref, w1c_ref, w2c_ref):
    i = pl.program_id(0)

    @pl.when((i == 0) | (te_ref[i] != te_ref[jnp.maximum(i - 1, 0)]))
    def _():
        w1c_ref[...] = w1_ref[0].astype(bf16)
        w2c_ref[...] = w2_ref[0].astype(bf16)

    rows = rows_ref[...].astype(bf16)
    he = jax.lax.dot_general(rows, w1c_ref[...], (((1,), (1,)), ((), ())),
                             preferred_element_type=f32)
    he = _gelu(he)
    o_ref[...] = jax.lax.dot_general(
        he.astype(bf16), w2c_ref[...], (((1,), (1,)), ((), ())),
        preferred_element_type=f32)


def _run_moe(te, hf_sorted, w1, w2):
    grid_spec = pltpu.PrefetchScalarGridSpec(
        num_scalar_prefetch=1,
        grid=(NT,),
        in_specs=[
            pl.BlockSpec((TS, D), lambda i, te_s: (i, 0)),
            pl.BlockSpec((1, F, D), lambda i, te_s: (te_s[i], 0, 0)),
            pl.BlockSpec((1, D, F), lambda i, te_s: (te_s[i], 0, 0)),
        ],
        out_specs=pl.BlockSpec((TS, D), lambda i, te_s: (i, 0)),
        scratch_shapes=[
            pltpu.VMEM((F, D), bf16),
            pltpu.VMEM((D, F), bf16),
        ],
    )
    return pl.pallas_call(
        _moe_kernel,
        grid_spec=grid_spec,
        out_shape=jax.ShapeDtypeStruct((NSLOT, D), f32),
        compiler_params=pltpu.CompilerParams(
            dimension_semantics=("arbitrary",)),
    )(te, hf_sorted, w1, w2)


def _gather_body(oe_hbm, dest_hbm, out_hbm, idx_v, rows_v, sem):
    wid = lax.axis_index("s") * 2 + lax.axis_index("c")
    base = wid * PPW
    pltpu.sync_copy(dest_hbm.at[pl.ds(base, PPW)], idx_v)
    pltpu.async_copy(oe_hbm.at[idx_v], rows_v, sem).wait()
    pltpu.sync_copy(rows_v, out_hbm.at[pl.ds(base, PPW)])


def _run_gather(oe_sorted, dest_flat):
    k = pl.kernel(
        _gather_body,
        mesh=_sc_mesh(),
        out_type=jax.ShapeDtypeStruct((NPAIR, D), f32),
        scratch_types=[
            pltpu.VMEM((PPW,), jnp.int32),
            pltpu.VMEM((PPW, D), f32),
            pltpu.SemaphoreType.DMA,
        ],
    )
    return k(oe_sorted, dest_flat)


def _combine_kernel(x2_ref, tw_ref, g0_ref, g1_ref, o_ref):
    tw = tw_ref[...]
    o_ref[...] = (x2_ref[...] + tw[:, 0:1] * g0_ref[...]
                  + tw[:, 1:2] * g1_ref[...])


def _run_combine(x2, tw, g01):
    return pl.pallas_call(
        _combine_kernel,
        grid=(S // T1,),
        in_specs=[
            pl.BlockSpec((T1, D), lambda i: (i, 0)),
            pl.BlockSpec((T1, 128), lambda i: (i, 0)),
            pl.BlockSpec((T1, D), lambda i: (i, 0)),
            pl.BlockSpec((T1, D), lambda i: (S // T1 + i, 0)),
        ],
        out_specs=pl.BlockSpec((T1, D), lambda i: (i, 0)),
        out_shape=jax.ShapeDtypeStruct((S, D), f32),
        compiler_params=pltpu.CompilerParams(
            dimension_semantics=("arbitrary",)),
    )(x2, tw, g01, g01)


def kernel(x, norm1_w, norm2_w, Wqkv, Wout, router_W, W1, W2):
    xs = x.reshape(S, D)
    n1 = norm1_w.reshape(1, D)
    n2 = norm2_w.reshape(1, D)
    wqkv_bf = Wqkv.astype(bf16)
    wout_bf = Wout.astype(bf16)
    rw_pad = jnp.zeros((128, D), f32).at[:E].set(router_W)

    qkv, tab = _run_qkv(xs, n1, wqkv_bf)
    ao = _run_attn(qkv, tab)
    x2, hf, tw, ti, aux = _run_post(xs, ao, wout_bf, n2, rw_pad)
    dest2, te2 = _run_meta(ti)
    dest_flat = jnp.concatenate([dest2[:, 0], dest2[:, 1]])
    te = te2[:NT, 0]
    hf_sorted = _run_dispatch(hf, dest_flat)
    oe_sorted = _run_moe(te, hf_sorted, W1, W2)
    g01 = _run_gather(oe_sorted, dest_flat)
    out = _run_combine(x2, tw, g01)
    return out.reshape(1, S, D), aux.reshape(())


# speedup vs baseline: 1.4011x; 1.0018x over previous
import jax
import jax.numpy as jnp
from jax import lax
from jax.experimental import pallas as pl
from jax.experimental.pallas import tpu as pltpu
from jax.experimental.pallas import tpu_sc as plsc

EPS = 1.1920929e-07
LOG_BASE = 9.210340371976184
S = 2048
D = 768
H = 12
DK = 64
E = 8
F = 2048
TQ = 512
T1 = 256
TS = 128
NPAIR = 2 * S
NSLOT = NPAIR + E * TS
NT = NSLOT // TS
NWORK = 32
PPW = NPAIR // NWORK
NEG = -1e30

bf16 = jnp.bfloat16
f32 = jnp.float32


def _rope2(x, cc, ss):
    partner = jnp.concatenate(
        [x[:, 32:64], x[:, :32], x[:, 96:128], x[:, 64:96]], axis=-1)
    return x * cc + partner * ss


def _gelu(x):
    return 0.5 * x * (1.0 + jax.lax.erf(x * 0.7071067811865476))


def _qkv_kernel(x_ref, n1_ref, w_ref, o_ref, tab_ref):
    x = x_ref[...]
    h = x * jax.lax.rsqrt(jnp.mean(x * x, axis=-1, keepdims=True) + EPS) * n1_ref[...]
    o_ref[...] = jax.lax.dot_general(
        h.astype(bf16), w_ref[...], (((1,), (1,)), ((), ())),
        preferred_element_type=f32).astype(bf16)
    pos = (pl.program_id(0) * T1
           + jax.lax.broadcasted_iota(jnp.int32, (T1, 32), 0)).astype(f32)
    fidx = jax.lax.broadcasted_iota(jnp.int32, (T1, 32), 1).astype(f32)
    ang = pos * jnp.exp(fidx * (-LOG_BASE / 32.0))
    c = jnp.cos(ang)
    sn = jnp.sin(ang)
    tab_ref[...] = jnp.concatenate([c, c, c, c, -sn, sn, -sn, sn], axis=-1)


def _run_qkv(xs, n1, wqkv_bf):
    return pl.pallas_call(
        _qkv_kernel,
        grid=(S // T1,),
        in_specs=[
            pl.BlockSpec((T1, D), lambda i: (i, 0)),
            pl.BlockSpec((1, D), lambda i: (0, 0)),
            pl.BlockSpec((3 * D, D), lambda i: (0, 0)),
        ],
        out_specs=[
            pl.BlockSpec((T1, 3 * D), lambda i: (i, 0)),
            pl.BlockSpec((T1, 256), lambda i: (i, 0)),
        ],
        out_shape=[
            jax.ShapeDtypeStruct((S, 3 * D), bf16),
            jax.ShapeDtypeStruct((S, 256), f32),
        ],
        compiler_params=pltpu.CompilerParams(
            dimension_semantics=("arbitrary",)),
    )(xs, n1, wqkv_bf)


TK = 512


def _attn_kernel(q_ref, k_ref, v_ref, tq_ref, tk_ref, o_ref, kr_ref):
    qt = pl.program_id(1)

    @pl.when(qt == 0)
    def _():
        k = k_ref[...].astype(f32)
        tk = tk_ref[...]
        kr_ref[...] = _rope2(k, tk[:, :128], tk[:, 128:]).astype(bf16)

    tq = tq_ref[...]
    q = _rope2(q_ref[...].astype(f32), tq[:, :128], tq[:, 128:]).astype(bf16)
    q0 = q[:, :DK]
    q1 = q[:, DK:]
    row = qt * TQ + jax.lax.broadcasted_iota(jnp.int32, (TQ, TK), 0)
    colb = jax.lax.broadcasted_iota(jnp.int32, (TQ, TK), 1)

    def body(kt, carry):
        o0, o1, l0, l1 = carry
        kc = kr_ref[pl.ds(kt * TK, TK), :]
        vc = v_ref[pl.ds(kt * TK, TK), :]
        col = kt * TK + colb

        def one(qh, hs):
            sc = jax.lax.dot_general(
                qh, kc[:, hs:hs + DK], (((1,), (1,)), ((), ())),
                preferred_element_type=f32)
            p = jnp.exp(jnp.where(col <= row, sc * 0.125 - 20.0, NEG))
            o = jax.lax.dot_general(
                p.astype(bf16), vc[:, hs:hs + DK], (((1,), (0,)), ((), ())),
                preferred_element_type=f32)
            return o, jnp.sum(p, axis=-1, keepdims=True)

        a0, b0 = one(q0, 0)
        a1, b1 = one(q1, DK)
        return o0 + a0, o1 + a1, l0 + b0, l1 + b1

    z = jnp.zeros((TQ, DK), f32)
    zl = jnp.zeros((TQ, 1), f32)
    o0, o1, l0, l1 = jax.lax.fori_loop(
        0, qt // (TK // TQ) + 1, body, (z, z, zl, zl))
    o_ref[...] = jnp.concatenate([o0 / l0, o1 / l1], axis=-1).astype(bf16)


def _run_attn(qkv, tab):
    return pl.pallas_call(
        _attn_kernel,
        grid=(H // 2, S // TQ),
        in_specs=[
            pl.BlockSpec((TQ, 2 * DK), lambda h, qt: (qt, h)),
            pl.BlockSpec((S, 2 * DK), lambda h, qt: (0, H // 2 + h)),
            pl.BlockSpec((S, 2 * DK), lambda h, qt: (0, H + h)),
            pl.BlockSpec((TQ, 256), lambda h, qt: (qt, 0)),
            pl.BlockSpec((S, 256), lambda h, qt: (0, 0)),
        ],
        out_specs=pl.BlockSpec((TQ, 2 * DK), lambda h, qt: (qt, h)),
        out_shape=jax.ShapeDtypeStruct((S, D), bf16),
        scratch_shapes=[pltpu.VMEM((S, 2 * DK), bf16)],
        compiler_params=pltpu.CompilerParams(
            dimension_semantics=("arbitrary", "arbitrary")),
    )(qkv, qkv, qkv, tab, tab)


def _post_kernel(x_ref, ao_ref, wo_ref, n2_ref, rw_ref,
                 x2_ref, hf_ref, tw_ref, ti_ref, aux_ref, ps_ref):
    tt = pl.program_id(0)
    x2 = x_ref[...] + jax.lax.dot_general(
        ao_ref[...], wo_ref[...], (((1,), (1,)), ((), ())),
        preferred_element_type=f32)
    x2_ref[...] = x2
    hf = x2 * jax.lax.rsqrt(jnp.mean(x2 * x2, axis=-1, keepdims=True) + EPS) * n2_ref[...]
    hf_ref[...] = hf
    logits = jax.lax.dot_general(hf, rw_ref[...], (((1,), (1,)), ((), ())),
                                 preferred_element_type=f32)
    colf = jax.lax.broadcasted_iota(jnp.int32, (T1, E), 1)
    m = jnp.max(logits, axis=-1, keepdims=True)
    p = jnp.exp(logits - m)
    probs = p / jnp.sum(p, axis=-1, keepdims=True)

    @pl.when(tt == 0)
    def _():
        ps_ref[...] = jnp.zeros_like(ps_ref)

    ps_ref[...] += jnp.sum(probs, axis=0, keepdims=True)

    @pl.when(tt == pl.num_programs(0) - 1)
    def _():
        mp = ps_ref[...] / jnp.float32(S)
        aux_ref[...] = jnp.sum(mp * mp, axis=-1, keepdims=True) * jnp.float32(E)

    m1 = jnp.max(probs, axis=-1, keepdims=True)
    i1 = jnp.min(jnp.where(probs == m1, colf, E), axis=-1, keepdims=True)
    probs2 = jnp.where(colf == i1, -1.0, probs)
    m2 = jnp.max(probs2, axis=-1, keepdims=True)
    i2 = jnp.min(jnp.where(probs2 == m2, colf, E), axis=-1, keepdims=True)
    tot = m1 + m2
    w1 = m1 / tot
    w2 = m2 / tot
    tw_ref[...] = (jnp.where(colf == 0, w1, 0.0)
                   + jnp.where(colf == 1, w2, 0.0))
    ti_ref[...] = (jnp.where(colf == 0, i1, 0)
                   + jnp.where(colf == 1, i2, 0))


def _run_post(xs, ao, wout_bf, n2, rw_pad):
    return pl.pallas_call(
        _post_kernel,
        grid=(S // T1,),
        in_specs=[
            pl.BlockSpec((T1, D), lambda i: (i, 0)),
            pl.BlockSpec((T1, D), lambda i: (i, 0)),
            pl.BlockSpec((D, D), lambda i: (0, 0)),
            pl.BlockSpec((1, D), lambda i: (0, 0)),
            pl.BlockSpec((E, D), lambda i: (0, 0)),
        ],
        out_specs=[
            pl.BlockSpec((T1, D), lambda i: (i, 0)),
            pl.BlockSpec((T1, D), lambda i: (i, 0)),
            pl.BlockSpec((T1, E), lambda i: (i, 0)),
            pl.BlockSpec((T1, E), lambda i: (i, 0)),
            pl.BlockSpec((1, 1), lambda i: (0, 0)),
        ],
        out_shape=[
            jax.ShapeDtypeStruct((S, D), f32),
            jax.ShapeDtypeStruct((S, D), f32),
            jax.ShapeDtypeStruct((S, E), f32),
            jax.ShapeDtypeStruct((S, E), jnp.int32),
            jax.ShapeDtypeStruct((1, 1), f32),
        ],
        scratch_shapes=[pltpu.VMEM((1, E), f32)],
        compiler_params=pltpu.CompilerParams(
            dimension_semantics=("arbitrary",)),
    )(xs, ao, wout_bf, n2, rw_pad)


def _cumsum(x, axis):
    n = x.shape[axis]
    k = 1
    while k < n:
        if axis == 0:
            pad = jnp.zeros((k, x.shape[1]), x.dtype)
            x = x + jnp.concatenate([pad, x[:-k]], axis=0)
        else:
            pad = jnp.zeros((x.shape[0], k), x.dtype)
            x = x + jnp.concatenate([pad, x[:, :-k]], axis=1)
        k *= 2
    return x


def _meta_kernel(ti_ref, dest_ref, te_ref):
    col = jax.lax.broadcasted_iota(jnp.int32, (S, E), 1)
    ti = ti_ref[...]
    i1 = ti[:, 0:1]
    i2 = ti[:, 1:2]
    oh1 = jnp.where(col == i1, 1, 0)
    oh2 = jnp.where(col == i2, 1, 0)
    c1 = _cumsum(oh1, 0)
    c2 = _cumsum(oh2, 0)
    cnt1 = c1[S - 1:S, :]
    cnt = cnt1 + c2[S - 1:S, :]
    padded = ((cnt + (TS - 1)) // TS) * TS
    offs = _cumsum(padded, 1) - padded
    d0 = jnp.sum(oh1 * (offs + c1 - 1), axis=1, keepdims=True)
    d1 = jnp.sum(oh2 * (offs + cnt1 + c2 - 1), axis=1, keepdims=True)
    dest_ref[...] = jnp.where(col == 0, d0, 0) + jnp.where(col == 1, d1, 0)
    rcol = jax.lax.broadcasted_iota(jnp.int32, (64, E), 1)
    rrow = jax.lax.broadcasted_iota(jnp.int32, (64, E), 0)
    ge = jnp.where(rrow * TS >= offs, 1, 0)
    te = jnp.sum(ge, axis=1, keepdims=True) - 1
    te_ref[...] = jnp.where(rcol == 0, te, 0)


def _run_meta(ti):
    return pl.pallas_call(
        _meta_kernel,
        grid=(1,),
        in_specs=[pl.BlockSpec((S, E), lambda i: (0, 0))],
        out_specs=[
            pl.BlockSpec((S, E), lambda i: (0, 0)),
            pl.BlockSpec((64, E), lambda i: (0, 0)),
        ],
        out_shape=[
            jax.ShapeDtypeStruct((S, E), jnp.int32),
            jax.ShapeDtypeStruct((64, E), jnp.int32),
        ],
        compiler_params=pltpu.CompilerParams(
            dimension_semantics=("arbitrary",)),
    )(ti)


def _sc_mesh():
    return plsc.VectorSubcoreMesh(core_axis_name="c", subcore_axis_name="s")


def _dispatch_body(hf_hbm, dest_hbm, out_hbm, idx_v, rows_v, sem):
    wid = lax.axis_index("s") * 2 + lax.axis_index("c")
    base = wid * PPW
    tok = lax.rem(base, S)
    pltpu.sync_copy(dest_hbm.at[pl.ds(base, PPW)], idx_v)
    pltpu.sync_copy(hf_hbm.at[pl.ds(tok, PPW)], rows_v)
    pltpu.async_copy(rows_v, out_hbm.at[idx_v], sem).wait()


def _run_dispatch(hf, dest_flat):
    k = pl.kernel(
        _dispatch_body,
        mesh=_sc_mesh(),
        out_type=jax.ShapeDtypeStruct((NSLOT, D), f32),
        scratch_types=[
            pltpu.VMEM((PPW,), jnp.int32),
            pltpu.VMEM((PPW, D), f32),
            pltpu.SemaphoreType.DMA,
        ],
    )
    return k(hf, dest_flat)


def _moe_kernel(te_ref, rows_ref, w1_ref, w2_ref, o_ref, w1c_ref, w2c_ref):
    i = pl.program_id(0)

    @pl.when((i == 0) | (te_ref[i] != te_ref[jnp.maximum(i - 1, 0)]))
    def _():
        w1c_ref[...] = w1_ref[0].astype(bf16)
        w2c_ref[...] = w2_ref[0].astype(bf16)

    rows = rows_ref[...].astype(bf16)
    he = jax.lax.dot_general(rows, w1c_ref[...], (((1,), (1,)), ((), ())),
                             preferred_element_type=f32)
    he = _gelu(he)
    o_ref[...] = jax.lax.dot_general(
        he.astype(bf16), w2c_ref[...], (((1,), (1,)), ((), ())),
        preferred_element_type=f32)


def _run_moe(te, hf_sorted, w1, w2):
    grid_spec = pltpu.PrefetchScalarGridSpec(
        num_scalar_prefetch=1,
        grid=(NT,),
        in_specs=[
            pl.BlockSpec((TS, D), lambda i, te_s: (i, 0)),
            pl.BlockSpec((1, F, D), lambda i, te_s: (te_s[i], 0, 0)),
            pl.BlockSpec((1, D, F), lambda i, te_s: (te_s[i], 0, 0)),
        ],
        out_specs=pl.BlockSpec((TS, D), lambda i, te_s: (i, 0)),
        scratch_shapes=[
            pltpu.VMEM((F, D), bf16),
            pltpu.VMEM((D, F), bf16),
        ],
    )
    return pl.pallas_call(
        _moe_kernel,
        grid_spec=grid_spec,
        out_shape=jax.ShapeDtypeStruct((NSLOT, D), f32),
        compiler_params=pltpu.CompilerParams(
            dimension_semantics=("arbitrary",)),
    )(te, hf_sorted, w1, w2)


def _gather_body(oe_hbm, dest_hbm, out_hbm, idx_v, rows_v, sem):
    wid = lax.axis_index("s") * 2 + lax.axis_index("c")
    base = wid * PPW
    pltpu.sync_copy(dest_hbm.at[pl.ds(base, PPW)], idx_v)
    pltpu.async_copy(oe_hbm.at[idx_v], rows_v, sem).wait()
    pltpu.sync_copy(rows_v, out_hbm.at[pl.ds(base, PPW)])


def _run_gather(oe_sorted, dest_flat):
    k = pl.kernel(
        _gather_body,
        mesh=_sc_mesh(),
        out_type=jax.ShapeDtypeStruct((NPAIR, D), f32),
        scratch_types=[
            pltpu.VMEM((PPW,), jnp.int32),
            pltpu.VMEM((PPW, D), f32),
            pltpu.SemaphoreType.DMA,
        ],
    )
    return k(oe_sorted, dest_flat)


def _combine_kernel(x2_ref, tw_ref, g0_ref, g1_ref, o_ref):
    tw = tw_ref[...]
    o_ref[...] = (x2_ref[...] + tw[:, 0:1] * g0_ref[...]
                  + tw[:, 1:2] * g1_ref[...])


def _run_combine(x2, tw, g01):
    return pl.pallas_call(
        _combine_kernel,
        grid=(S // T1,),
        in_specs=[
            pl.BlockSpec((T1, D), lambda i: (i, 0)),
            pl.BlockSpec((T1, E), lambda i: (i, 0)),
            pl.BlockSpec((T1, D), lambda i: (i, 0)),
            pl.BlockSpec((T1, D), lambda i: (S // T1 + i, 0)),
        ],
        out_specs=pl.BlockSpec((T1, D), lambda i: (i, 0)),
        out_shape=jax.ShapeDtypeStruct((S, D), f32),
        compiler_params=pltpu.CompilerParams(
            dimension_semantics=("arbitrary",)),
    )(x2, tw, g01, g01)


def kernel(x, norm1_w, norm2_w, Wqkv, Wout, router_W, W1, W2):
    xs = x.reshape(S, D)
    n1 = norm1_w.reshape(1, D)
    n2 = norm2_w.reshape(1, D)
    wqkv_bf = Wqkv.astype(bf16)
    wout_bf = Wout.astype(bf16)

    qkv, tab = _run_qkv(xs, n1, wqkv_bf)
    ao = _run_attn(qkv, tab)
    x2, hf, tw, ti, aux = _run_post(xs, ao, wout_bf, n2, router_W)
    dest2, te2 = _run_meta(ti)
    dest_flat = jnp.concatenate([dest2[:, 0], dest2[:, 1]])
    te = te2[:NT, 0]
    hf_sorted = _run_dispatch(hf, dest_flat)
    oe_sorted = _run_moe(te, hf_sorted, W1, W2)
    g01 = _run_gather(oe_sorted, dest_flat)
    out = _run_combine(x2, tw, g01)
    return out.reshape(1, S, D), aux.reshape(())
